# baseline (speedup 1.0000x reference)
import functools

import jax
import jax.numpy as jnp
from jax import lax
from jax.experimental import pallas as pl
from jax.experimental.pallas import tpu as pltpu
from jax.experimental.pallas import tpu_sc as plsc

NUM_CLASSES = 100000
V = NUM_CLASSES + 1
HIDDEN = 64
BATCH = 16384

NC = 2
NS = 16
L = 16
NW = NC * NS

RANGE = 3584
CW = 256
NCHUNK = RANGE // CW
WIN = 2048
NWIN = BATCH // WIN
HCAP = 1040
CCAP = 256
SCAP = 512
OPAD = 128


def _sc_body(labels_hbm, fd_hbm, tt_hbm, out_hbm,
             lw0, fw0, lw1, fw1,
             hla, hpa, hlb, hpb,
             binl, binp,
             cnt16,
             cba, cbb,
             stag,
             sem_w0, sem_w1, sem_c, sem_o):
    wid = lax.axis_index("s") * NC + lax.axis_index("c")
    c0 = wid * RANGE
    iota16 = lax.iota(jnp.int32, L)
    zerov = jnp.zeros((L,), jnp.int32)

    def fire_chunk(k, buf):
        cstart = c0 + k * CW

        @pl.when(jnp.logical_and(cstart < V, k < NCHUNK))
        def _():
            pltpu.async_copy(tt_hbm.at[:, pl.ds(cstart, CW)], buf, sem_c)

    def wait_chunk(k, buf):
        cstart = c0 + k * CW

        @pl.when(jnp.logical_and(cstart < V, k < NCHUNK))
        def _():
            pltpu.make_async_copy(tt_hbm.at[:, pl.ds(cstart, CW)],
                                  buf, sem_c).wait()

    fire_chunk(jnp.int32(0), cba)

    def fire_win(w):
        lw, fw, sw = (lw0, fw0, sem_w0) if w % 2 == 0 else (lw1, fw1, sem_w1)
        pltpu.async_copy(labels_hbm.at[pl.ds(w * WIN, WIN)], lw, sw)
        pltpu.async_copy(fd_hbm.at[pl.ds(w * WIN, WIN)], fw, sw)

    def wait_win(w):
        lw, fw, sw = (lw0, fw0, sem_w0) if w % 2 == 0 else (lw1, fw1, sem_w1)
        pltpu.make_async_copy(labels_hbm.at[pl.ds(w * WIN, WIN)], lw, sw).wait()
        pltpu.make_async_copy(fd_hbm.at[pl.ds(w * WIN, WIN)], fw, sw).wait()

    fire_win(0)
    scan_scope = jax.named_scope("scan_phase")
    scan_scope.__enter__()
    cnts = (zerov, zerov)
    for w in range(NWIN):
        wait_win(w)
        if w + 1 < NWIN:
            fire_win(w + 1)
        lw, fw = (lw0, fw0) if w % 2 == 0 else (lw1, fw1)

        def scan_body(i, cur, lw=lw, fw=fw, w=w):
            new = []
            for half, (lst, pst) in enumerate(((hla, hpa), (hlb, hpb))):
                sl = pl.ds((2 * i + half) * L, L)
                m = jnp.where(fw[sl] == 1, jnp.int32(NUM_CLASSES), lw[sl])
                rel = m - c0
                hit = rel.astype(jnp.uint32) < jnp.uint32(RANGE)
                d = plsc.all_reduce_population_count(hit)
                slot = cur[half] + plsc.cumsum(hit.astype(jnp.int32)) - 1
                plsc.store_scatter(lst, [slot], rel, mask=hit)
                pos = iota16 + (w * WIN + (2 * i + half) * L)
                plsc.store_scatter(pst, [slot], pos, mask=hit)
                new.append(cur[half] + d)
            return tuple(new)

        cnts = lax.fori_loop(0, WIN // (2 * L), scan_body, cnts)
    cnta, cntb = cnts[0][0], cnts[1][0]
    scan_scope.__exit__(None, None, None)

    cnt16[pl.ds(0, L)] = zerov

    def bin_list(lst, pst, cn):
        def body(i, carry):
            rel = lst[pl.ds(i * L, L)]
            p = pst[pl.ds(i * L, L)]
            valid = (iota16 + i * L) < cn
            cid = jnp.where(valid, jnp.right_shift(rel, 8), jnp.int32(15))
            base = plsc.load_gather(cnt16, [cid])
            rank, lastm = plsc.scan_count(cid)
            slot = base + rank - 1
            lcol = rel - (cid << 8)
            idx = cid * CCAP + slot
            plsc.store_scatter(binl, [idx], lcol, mask=valid)
            plsc.store_scatter(binp, [idx], p, mask=valid)
            plsc.store_scatter(cnt16, [cid], base + rank, mask=lastm)
            return carry

        lax.fori_loop(0, (cn + (L - 1)) // L, body, 0)

    with jax.named_scope("bin_phase"):
        bin_list(hla, hpa, cnta)
        bin_list(hlb, hpb, cntb)

    def gather_group(src, cols, srows):
        for j64 in range(HIDDEN):
            jv = zerov + j64
            vals = plsc.load_gather(src, [jv, cols])
            plsc.store_scatter(stag, [srows, jv], vals)

    def extract(kbase, src, ck):
        nfull = ck // L

        def group(g, carry):
            cols = binl[pl.ds(kbase + g * L, L)]
            gather_group(src, cols, iota16 + g * L)
            poss = binp[pl.ds(kbase + g * L, L)]
            for j in range(L):
                pltpu.async_copy(stag.at[g * L + j], out_hbm.at[poss[j]], sem_o)
            return carry

        lax.fori_loop(0, nfull, group, 0)

        @pl.when(nfull * L < ck)
        def _():
            g = nfull
            valid = (iota16 + g * L) < ck
            cols = jnp.where(valid, binl[pl.ds(kbase + g * L, L)], 0)
            gather_group(src, cols, iota16 + g * L)
            poss = binp[pl.ds(kbase + g * L, L)]
            for j in range(L):
                @pl.when(g * L + j < ck)
                def _(j=j, g=g, poss=poss):
                    pltpu.async_copy(stag.at[g * L + j], out_hbm.at[poss[j]], sem_o)

    def drain(ck):
        def dg(i, carry):
            pltpu.make_async_copy(out_hbm.at[pl.ds(0, L)],
                                  stag.at[pl.ds(0, L)], sem_o).wait()
            return carry

        lax.fori_loop(0, ck // L, dg, 0)

        def dr(i, carry):
            pltpu.make_async_copy(out_hbm.at[0], stag.at[0], sem_o).wait()
            return carry

        lax.fori_loop(0, ck - (ck // L) * L, dr, 0)

    def block_count(k):
        return plsc.load_gather(cnt16, [zerov + k])[0]

    def chunk_pair(t, carry):
        k0 = t * 2
        k1 = k0 + 1
        ck = block_count(k0)
        with jax.named_scope("cwait"):
            wait_chunk(k0, cba)
        fire_chunk(k1, cbb)
        with jax.named_scope("egroups"):
            extract(k0 * CCAP, cba, ck)
        with jax.named_scope("edrain"):
            drain(ck)
        ck = block_count(k1)
        with jax.named_scope("cwait"):
            wait_chunk(k1, cbb)
        fire_chunk(k0 + 2, cba)
        with jax.named_scope("egroups"):
            extract(k1 * CCAP, cbb, ck)
        with jax.named_scope("edrain"):
            drain(ck)
        return carry

    with jax.named_scope("extract_phase"):
        lax.fori_loop(0, NCHUNK // 2, chunk_pair, jnp.int32(0))

@functools.lru_cache(maxsize=1)
def _build():
    mesh = plsc.VectorSubcoreMesh(
        core_axis_name="c", subcore_axis_name="s",
        num_cores=NC, num_subcores=NS)
    return pl.kernel(
        _sc_body,
        out_type=jax.ShapeDtypeStruct((BATCH, HIDDEN), jnp.float32),
        mesh=mesh,
        scratch_types=[
            pltpu.VMEM((WIN,), jnp.int32),
            pltpu.VMEM((WIN,), jnp.int32),
            pltpu.VMEM((WIN,), jnp.int32),
            pltpu.VMEM((WIN,), jnp.int32),
            pltpu.VMEM((HCAP,), jnp.int32),
            pltpu.VMEM((HCAP,), jnp.int32),
            pltpu.VMEM((HCAP,), jnp.int32),
            pltpu.VMEM((HCAP,), jnp.int32),
            pltpu.VMEM((16 * CCAP,), jnp.int32),
            pltpu.VMEM((16 * CCAP,), jnp.int32),
            pltpu.VMEM((L,), jnp.int32),
            pltpu.VMEM((HIDDEN, CW), jnp.float32),
            pltpu.VMEM((HIDDEN, CW), jnp.float32),
            pltpu.VMEM((SCAP, HIDDEN), jnp.float32),
            pltpu.SemaphoreType.DMA,
            pltpu.SemaphoreType.DMA,
            pltpu.SemaphoreType.DMA,
            pltpu.SemaphoreType.DMA,
        ],
        compiler_params=pltpu.CompilerParams(
            use_tc_tiling_on_sc=True, needs_layout_passes=False),
    )


def kernel(labels, train, force_drop_ids, embedding_table):
    del train
    return _build()(labels, force_drop_ids, embedding_table.T)


# speedup vs baseline: 1.2577x; 1.2577x over previous
import functools

import jax
import jax.numpy as jnp
from jax import lax
from jax.experimental import pallas as pl
from jax.experimental.pallas import tpu as pltpu
from jax.experimental.pallas import tpu_sc as plsc

NUM_CLASSES = 100000
HIDDEN = 64
BATCH = 16384

NC = 2
NS = 16
L = 16
NW = NC * NS
CHUNK = BATCH // NW


def _sc_body(labels_hbm, fd_hbm, table_hbm, out_hbm, lbl_v, fd_v, rows_v, sem):
    wid = lax.axis_index("s") * NC + lax.axis_index("c")
    base = wid * CHUNK

    pltpu.sync_copy(labels_hbm.at[pl.ds(base, CHUNK)], lbl_v)
    pltpu.sync_copy(fd_hbm.at[pl.ds(base, CHUNK)], fd_v)

    def fire_group(g, carry):
        sl = pl.ds(g * L, L)
        lbl = lbl_v[sl]
        fd = fd_v[sl]
        idx = jnp.where(fd == 1, jnp.int32(NUM_CLASSES), lbl)
        for j in range(L):
            pltpu.async_copy(table_hbm.at[idx[j]], rows_v.at[g * L + j], sem)
        return carry

    lax.fori_loop(0, CHUNK // L, fire_group, 0)

    pltpu.make_async_copy(table_hbm.at[pl.ds(0, CHUNK)], rows_v, sem).wait()

    pltpu.sync_copy(rows_v, out_hbm.at[pl.ds(base, CHUNK)])


@functools.lru_cache(maxsize=1)
def _build():
    mesh = plsc.VectorSubcoreMesh(
        core_axis_name="c", subcore_axis_name="s",
        num_cores=NC, num_subcores=NS)
    return pl.kernel(
        _sc_body,
        out_type=jax.ShapeDtypeStruct((BATCH, HIDDEN), jnp.float32),
        mesh=mesh,
        scratch_types=[
            pltpu.VMEM((CHUNK,), jnp.int32),
            pltpu.VMEM((CHUNK,), jnp.int32),
            pltpu.VMEM((CHUNK, HIDDEN), jnp.float32),
            pltpu.SemaphoreType.DMA,
        ],
        compiler_params=pltpu.CompilerParams(use_tc_tiling_on_sc=True),
    )


def kernel(labels, train, force_drop_ids, embedding_table):
    del train
    return _build()(labels, force_drop_ids, embedding_table)
